# baseline (speedup 1.0000x reference)
import functools

import jax
import jax.numpy as jnp
from jax.experimental import pallas as pl
from jax.experimental.pallas import tpu as pltpu

T = 256
N = 16
D = 512
H = 512
TS = 64
LANE = 128
UNROLL = 32


def _gru_gate(h, m, gh_unbiased, bhh, gi_b, h_dim):
    gi = gi_b.astype(jnp.float32)
    gh = gh_unbiased + bhh
    r = jax.nn.sigmoid(gi[:, :h_dim] + gh[:, :h_dim])
    z = jax.nn.sigmoid(gi[:, h_dim:2 * h_dim] + gh[:, h_dim:2 * h_dim])
    cand = jnp.tanh(gi[:, 2 * h_dim:] + r * gh[:, 2 * h_dim:])
    return (1.0 - z) * cand + z * (h * m)


def _gru2_kernel(x_ref, wih0_ref, whh0_ref, bih0_ref, bhh0_ref,
                 wih1_ref, whh1_ref, bih1_ref, bhh1_ref, m0_ref, m1_ref,
                 hinit_ref, out_ref, hn_ref,
                 h0_ref, h1_ref, gi0_ref, mid_ref, gi1_ref,
                 *, ts, n, h_dim, nchunks):
    c = pl.program_id(0)

    @pl.when(c == 0)
    def _():
        h0_ref[...] = hinit_ref[0]
        h1_ref[...] = hinit_ref[1]

    @pl.when(c < nchunks)
    def _():
        xc = x_ref[...].reshape(ts * n, x_ref.shape[2])
        gi0_ref[...] = (
            jnp.dot(xc, wih0_ref[...], preferred_element_type=jnp.float32)
            + bih0_ref[0:1, :]
        ).astype(jnp.bfloat16)

    def l0_step(i, h0):
        m0 = m0_ref[pl.ds(i, 1), :, :].reshape(n, LANE)[:, 0:1]
        gh0 = jnp.dot((h0 * m0).astype(jnp.bfloat16), whh0_ref[...],
                      preferred_element_type=jnp.float32)
        h0n = _gru_gate(h0, m0, gh0, bhh0_ref[0:1, :],
                        gi0_ref[pl.ds(i * n, n), :], h_dim)
        mid_ref[pl.ds(i * n, n), :] = h0n.astype(jnp.bfloat16)
        return h0n

    def l1_step(i, h1):
        m1 = m1_ref[pl.ds(i, 1), :, :].reshape(n, LANE)[:, 0:1]
        gh1 = jnp.dot((h1 * m1).astype(jnp.bfloat16), whh1_ref[...],
                      preferred_element_type=jnp.float32)
        h1n = _gru_gate(h1, m1, gh1, bhh1_ref[0:1, :],
                        gi1_ref[pl.ds(i * n, n), :], h_dim)
        out_ref[pl.ds(i, 1)] = h1n.reshape(1, n, h_dim)
        return h1n

    def joint_step(i, carry):
        h0, h1 = carry
        return (l0_step(i, h0), l1_step(i, h1))

    @pl.when(c == 0)
    def _():
        h0_ref[...] = jax.lax.fori_loop(0, ts, l0_step, h0_ref[...],
                                        unroll=UNROLL)

    @pl.when(jnp.logical_and(c > 0, c < nchunks))
    def _():
        h0, h1 = jax.lax.fori_loop(0, ts, joint_step,
                                   (h0_ref[...], h1_ref[...]),
                                   unroll=UNROLL)
        h0_ref[...] = h0
        h1_ref[...] = h1

    @pl.when(c == nchunks)
    def _():
        h1_ref[...] = jax.lax.fori_loop(0, ts, l1_step, h1_ref[...],
                                        unroll=UNROLL)

    @pl.when(c == nchunks - 1)
    def _():
        hn_ref[0] = h0_ref[...]

    @pl.when(c == nchunks)
    def _():
        hn_ref[1] = h1_ref[...]

    @pl.when(c < nchunks)
    def _():
        gi1_ref[...] = (
            jnp.dot(mid_ref[...], wih1_ref[...],
                    preferred_element_type=jnp.float32)
            + bih1_ref[0:1, :]
        ).astype(jnp.bfloat16)


def kernel(x, hidden_states, masks, W_ih0, W_hh0, b_ih0, b_hh0,
           W_ih1, W_hh1, b_ih1, b_hh1):
    xs = x.reshape(T, N, D).astype(jnp.bfloat16)
    masks_b = jnp.broadcast_to(
        masks.astype(jnp.float32).reshape(T, N, 1), (T, N, LANE))

    def prep_b(b):
        return jnp.broadcast_to(b.reshape(1, 3 * H), (8, 3 * H))

    nchunks = T // TS
    last = nchunks - 1
    body = functools.partial(_gru2_kernel, ts=TS, n=N, h_dim=H,
                             nchunks=nchunks)
    full = lambda shape: pl.BlockSpec(shape, lambda c: (0,) * len(shape))
    out1, h_n = pl.pallas_call(
        body,
        grid=(nchunks + 1,),
        in_specs=[
            pl.BlockSpec((TS, N, D), lambda c: (jnp.minimum(c, last), 0, 0)),
            full((D, 3 * H)),
            full((H, 3 * H)),
            full((8, 3 * H)),
            full((8, 3 * H)),
            full((H, 3 * H)),
            full((H, 3 * H)),
            full((8, 3 * H)),
            full((8, 3 * H)),
            pl.BlockSpec((TS, N, LANE),
                         lambda c: (jnp.minimum(c, last), 0, 0)),
            pl.BlockSpec((TS, N, LANE),
                         lambda c: (jnp.maximum(c - 1, 0), 0, 0)),
            full((2, N, H)),
        ],
        out_specs=[
            pl.BlockSpec((TS, N, H), lambda c: (jnp.maximum(c - 1, 0), 0, 0)),
            full((2, N, H)),
        ],
        out_shape=[
            jax.ShapeDtypeStruct((T, N, H), jnp.float32),
            jax.ShapeDtypeStruct((2, N, H), jnp.float32),
        ],
        scratch_shapes=[
            pltpu.VMEM((N, H), jnp.float32),
            pltpu.VMEM((N, H), jnp.float32),
            pltpu.VMEM((TS * N, 3 * H), jnp.bfloat16),
            pltpu.VMEM((TS * N, H), jnp.bfloat16),
            pltpu.VMEM((TS * N, 3 * H), jnp.bfloat16),
        ],
        compiler_params=pltpu.CompilerParams(
            dimension_semantics=("arbitrary",),
        ),
    )(xs, W_ih0.T.astype(jnp.bfloat16), W_hh0.T.astype(jnp.bfloat16),
      prep_b(b_ih0), prep_b(b_hh0),
      W_ih1.T.astype(jnp.bfloat16), W_hh1.T.astype(jnp.bfloat16),
      prep_b(b_ih1), prep_b(b_hh1),
      masks_b, masks_b, hidden_states)

    return out1.reshape(T * N, H), h_n
